# Initial kernel scaffold; baseline (speedup 1.0000x reference)
import functools

import jax
import jax.numpy as jnp
from jax import lax
from jax.experimental import pallas as pl
from jax.experimental.pallas import tpu as pltpu
from jax.experimental.pallas import tpu_sc as plsc

_NC = 2
_NS = 16
_NW = _NC * _NS

_LANES = 16
_BAGS_PER_GROUP = 2


def _sc_bag_sums_body(groups_per_w, bag, emb, table_hbm, text_hbm, out_hbm,
                      idx_v, rows0, rows1, out_v, sem0, sem1):
    bags_per_w = groups_per_w * _BAGS_PER_GROUP
    cols = emb // _LANES

    wid = lax.axis_index("s") * _NC + lax.axis_index("c")

    pltpu.sync_copy(text_hbm.at[pl.ds(wid * groups_per_w, groups_per_w)], idx_v)

    def gather(g, buf, sem):
        return pltpu.make_async_copy(table_hbm.at[idx_v.at[g]], buf, sem)

    def reduce_group(g, buf):
        for b in range(_BAGS_PER_GROUP):
            r0 = b * bag
            accs = [buf[r0, pl.ds(c * _LANES, _LANES)] for c in range(cols)]
            for i in range(1, bag):
                for c in range(cols):
                    accs[c] = accs[c] + buf[r0 + i, pl.ds(c * _LANES, _LANES)]
            row = _BAGS_PER_GROUP * g + b
            for c in range(cols):
                out_v[row, pl.ds(c * _LANES, _LANES)] = accs[c]

    gather(0, rows0, sem0).start()

    def body(gg, carry):
        g0 = 2 * gg
        g1 = g0 + 1
        gather(g1, rows1, sem1).start()
        gather(g0, rows0, sem0).wait()
        reduce_group(g0, rows0)

        @pl.when(g1 + 1 < groups_per_w)
        def _():
            gather(g1 + 1, rows0, sem0).start()

        gather(g1, rows1, sem1).wait()
        reduce_group(g1, rows1)
        return carry

    lax.fori_loop(0, groups_per_w // 2, body, 0)

    pltpu.sync_copy(out_v, out_hbm.at[pl.ds(wid * bags_per_w, bags_per_w)])


def _mlp_body(inv_bag, x_ref, w1_ref, b1_ref, w2_ref, b2_ref, o_ref):
    x = x_ref[...] * inv_bag
    h = jnp.maximum(
        jnp.dot(x, w1_ref[...], preferred_element_type=jnp.float32)
        + b1_ref[...], 0.0)
    o_ref[...] = (
        jnp.dot(h, w2_ref[...], preferred_element_type=jnp.float32)
        + b2_ref[...])


def kernel(text, table, W1, b1, W2, b2):
    B, bag = text.shape
    V, emb = table.shape
    H = W1.shape[1]
    C = W2.shape[1]

    bags_per_w = B // _NW
    groups_per_w = bags_per_w // _BAGS_PER_GROUP
    idx_per_group = _BAGS_PER_GROUP * bag

    text2 = text.astype(jnp.int32).reshape(_NW * groups_per_w, idx_per_group)

    mesh = plsc.VectorSubcoreMesh(core_axis_name="c", subcore_axis_name="s")
    sums = pl.kernel(
        functools.partial(_sc_bag_sums_body, groups_per_w, bag, emb),
        out_type=jax.ShapeDtypeStruct((B, emb), jnp.float32),
        mesh=mesh,
        scratch_types=[
            pltpu.VMEM((groups_per_w, idx_per_group), jnp.int32),
            pltpu.VMEM((idx_per_group, emb), jnp.float32),
            pltpu.VMEM((idx_per_group, emb), jnp.float32),
            pltpu.VMEM((bags_per_w, emb), jnp.float32),
            pltpu.SemaphoreType.DMA,
            pltpu.SemaphoreType.DMA,
        ],
    )(table, text2)

    BM = 1024
    logits = pl.pallas_call(
        functools.partial(_mlp_body, 1.0 / bag),
        grid=(B // BM,),
        in_specs=[
            pl.BlockSpec((BM, emb), lambda i: (i, 0)),
            pl.BlockSpec((emb, H), lambda i: (0, 0)),
            pl.BlockSpec((1, H), lambda i: (0, 0)),
            pl.BlockSpec((H, C), lambda i: (0, 0)),
            pl.BlockSpec((1, C), lambda i: (0, 0)),
        ],
        out_specs=pl.BlockSpec((BM, C), lambda i: (i, 0)),
        out_shape=jax.ShapeDtypeStruct((B, C), jnp.float32),
    )(sums, W1, b1.reshape(1, H), W2, b2.reshape(1, C))
    return logits


# speedup vs baseline: 2.5120x; 2.5120x over previous
import functools

import jax
import jax.numpy as jnp
from jax import lax
from jax.experimental import pallas as pl
from jax.experimental.pallas import tpu as pltpu
from jax.experimental.pallas import tpu_sc as plsc

_NC = 2
_NS = 16
_NW = _NC * _NS

_LANES = 16
_BAGS_PER_GROUP = 2


def _sc_bag_sums_body(groups_per_w, bag, emb, table_hbm, text_hbm, out_hbm,
                      idx_v, rows0, rows1, out_v, sem0, sem1):
    bags_per_w = groups_per_w * _BAGS_PER_GROUP
    cols = emb // _LANES

    wid = lax.axis_index("s") * _NC + lax.axis_index("c")

    pltpu.sync_copy(text_hbm.at[pl.ds(wid * groups_per_w, groups_per_w)], idx_v)

    def gather(g, buf, sem):
        return pltpu.make_async_copy(table_hbm.at[idx_v.at[g]], buf, sem)

    def reduce_group(g, buf):
        for b in range(_BAGS_PER_GROUP):
            r0 = b * bag
            accs = [buf[r0, pl.ds(c * _LANES, _LANES)] for c in range(cols)]
            for i in range(1, bag):
                for c in range(cols):
                    accs[c] = accs[c] + buf[r0 + i, pl.ds(c * _LANES, _LANES)]
            row = _BAGS_PER_GROUP * g + b
            for c in range(cols):
                out_v[row, pl.ds(c * _LANES, _LANES)] = accs[c]

    gather(0, rows0, sem0).start()

    def body(gg, carry):
        g0 = 2 * gg
        g1 = g0 + 1
        gather(g1, rows1, sem1).start()
        gather(g0, rows0, sem0).wait()
        reduce_group(g0, rows0)

        @pl.when(g1 + 1 < groups_per_w)
        def _():
            gather(g1 + 1, rows0, sem0).start()

        gather(g1, rows1, sem1).wait()
        reduce_group(g1, rows1)
        return carry

    lax.fori_loop(0, groups_per_w // 2, body, 0)

    pltpu.sync_copy(out_v, out_hbm.at[pl.ds(wid * bags_per_w, bags_per_w)])


def _mlp_body(inv_bag, x_ref, w1_ref, b1_ref, w2_ref, b2_ref, o_ref):
    x = x_ref[...] * inv_bag
    h = jnp.maximum(
        jnp.dot(x, w1_ref[...], preferred_element_type=jnp.float32)
        + b1_ref[...], 0.0)
    o_ref[...] = (
        jnp.dot(h, w2_ref[...], preferred_element_type=jnp.float32)
        + b2_ref[...])


def kernel(text, table, W1, b1, W2, b2):
    B, bag = text.shape
    V, emb = table.shape
    H = W1.shape[1]
    C = W2.shape[1]

    bags_per_w = B // _NW
    groups_per_w = bags_per_w // _BAGS_PER_GROUP
    idx_per_group = _BAGS_PER_GROUP * bag

    text2 = text.astype(jnp.int32).reshape(_NW * groups_per_w, idx_per_group)

    mesh = plsc.VectorSubcoreMesh(core_axis_name="c", subcore_axis_name="s")
    sums = pl.kernel(
        functools.partial(_sc_bag_sums_body, groups_per_w, bag, emb),
        out_type=jax.ShapeDtypeStruct((B, emb), jnp.float32),
        mesh=mesh,
        compiler_params=pltpu.CompilerParams(use_tc_tiling_on_sc=False),
        scratch_types=[
            pltpu.VMEM((groups_per_w, idx_per_group), jnp.int32),
            pltpu.VMEM((idx_per_group, emb), jnp.float32),
            pltpu.VMEM((idx_per_group, emb), jnp.float32),
            pltpu.VMEM((bags_per_w, emb), jnp.float32),
            pltpu.SemaphoreType.DMA,
            pltpu.SemaphoreType.DMA,
        ],
    )(table, text2)

    BM = 1024
    logits = pl.pallas_call(
        functools.partial(_mlp_body, 1.0 / bag),
        grid=(B // BM,),
        in_specs=[
            pl.BlockSpec((BM, emb), lambda i: (i, 0)),
            pl.BlockSpec((emb, H), lambda i: (0, 0)),
            pl.BlockSpec((1, H), lambda i: (0, 0)),
            pl.BlockSpec((H, C), lambda i: (0, 0)),
            pl.BlockSpec((1, C), lambda i: (0, 0)),
        ],
        out_specs=pl.BlockSpec((BM, C), lambda i: (i, 0)),
        out_shape=jax.ShapeDtypeStruct((B, C), jnp.float32),
    )(sums, W1, b1.reshape(1, H), W2, b2.reshape(1, C))
    return logits
